# Initial kernel scaffold; baseline (speedup 1.0000x reference)
import functools

import jax
import jax.numpy as jnp
from jax import lax
from jax.experimental import pallas as pl
from jax.experimental.pallas import tpu as pltpu
from jax.experimental.pallas import tpu_sc as plsc

N = 10000
E = 320000
D = 16
B = 16384

NC = 2
NS = 16
NW = NC * NS
CHUNK = 128

ECH = -(-E // (NW * CHUNK))
EW = ECH * CHUNK
EPAD = NW * EW

NPAD = 8
NT = N + NPAD
NDEG = 10240

PCH = (2 * B) // (NW * CHUNK)

_mesh = plsc.VectorSubcoreMesh(
    core_axis_name="c", subcore_axis_name="s", num_cores=NC, num_subcores=NS
)


@functools.partial(
    pl.kernel,
    out_type=jax.ShapeDtypeStruct((NC, NDEG), jnp.float32),
    mesh=_mesh,
    scratch_types=[
        pltpu.VMEM((ECH, CHUNK), jnp.int32),
        pltpu.VMEM((CHUNK,), jnp.float32),
        pltpu.VMEM_SHARED((NDEG,), jnp.float32),
    ],
)
def _deg_kernel(dst_hbm, ones_hbm, zeros_hbm, deg_out, dst_v, ones_v, shared_deg):
    cid = lax.axis_index("c")
    sid = lax.axis_index("s")
    wid = cid * NS + sid

    @pl.when(sid == 0)
    def _():
        pltpu.sync_copy(zeros_hbm, shared_deg)

    pltpu.sync_copy(ones_hbm, ones_v)
    pltpu.sync_copy(dst_hbm.at[wid], dst_v)
    plsc.subcore_barrier()

    @pl.loop(0, ECH)
    def _(k):
        pltpu.sync_copy(ones_v, shared_deg.at[dst_v.at[k]], add=True)

    plsc.subcore_barrier()

    @pl.when(sid == 0)
    def _():
        pltpu.sync_copy(shared_deg, deg_out.at[cid])


@functools.partial(
    pl.kernel,
    out_type=jax.ShapeDtypeStruct((NC, NT, D), jnp.float32),
    mesh=_mesh,
    scratch_types=[
        pltpu.VMEM((ECH, CHUNK), jnp.int32),
        pltpu.VMEM((ECH, CHUNK), jnp.int32),
        pltpu.VMEM((CHUNK, D), jnp.float32),
        pltpu.VMEM_SHARED((NT, D), jnp.float32),
    ],
)
def _edge_kernel(y_hbm, src_hbm, dst_hbm, zeros_hbm, acc_out,
                 src_v, dst_v, rows_v, shared_acc):
    cid = lax.axis_index("c")
    sid = lax.axis_index("s")
    wid = cid * NS + sid

    @pl.when(sid == 0)
    def _():
        pltpu.sync_copy(zeros_hbm, shared_acc)

    pltpu.sync_copy(src_hbm.at[wid], src_v)
    pltpu.sync_copy(dst_hbm.at[wid], dst_v)
    plsc.subcore_barrier()

    @pl.loop(0, ECH)
    def _(k):
        pltpu.sync_copy(y_hbm.at[src_v.at[k]], rows_v)
        pltpu.sync_copy(rows_v, shared_acc.at[dst_v.at[k]], add=True)

    plsc.subcore_barrier()

    @pl.when(sid == 0)
    def _():
        pltpu.sync_copy(shared_acc, acc_out.at[cid])


@functools.partial(
    pl.kernel,
    out_type=jax.ShapeDtypeStruct((2 * B, 2 * D), jnp.float32),
    mesh=_mesh,
    scratch_types=[
        pltpu.VMEM((PCH, CHUNK), jnp.int32),
        pltpu.VMEM((PCH * CHUNK, 2 * D), jnp.float32),
    ],
)
def _pair_gather_kernel(ext_hbm, idx_hbm, rows_out, idx_v, rows_v):
    cid = lax.axis_index("c")
    sid = lax.axis_index("s")
    wid = cid * NS + sid

    pltpu.sync_copy(idx_hbm.at[wid], idx_v)

    @pl.loop(0, PCH)
    def _(k):
        pltpu.sync_copy(ext_hbm.at[idx_v.at[k]], rows_v.at[pl.ds(k * CHUNK, CHUNK)])

    pltpu.sync_copy(rows_v, rows_out.at[pl.ds(wid * (PCH * CHUNK), PCH * CHUNK)])


def _mm_body(a_ref, w_ref, o_ref):
    o_ref[...] = jnp.dot(a_ref[...], w_ref[...], preferred_element_type=jnp.float32)


def _xw(features, gcn_weight):
    rows = 400
    return pl.pallas_call(
        _mm_body,
        grid=(N // rows,),
        in_specs=[
            pl.BlockSpec((rows, N), lambda i: (i, 0)),
            pl.BlockSpec((N, D), lambda i: (0, 0)),
        ],
        out_specs=pl.BlockSpec((rows, D), lambda i: (i, 0)),
        out_shape=jax.ShapeDtypeStruct((N, D), jnp.float32),
    )(features, gcn_weight)


def _y_body(x_ref, da_ref, db_ref, y_ref):
    deg = da_ref[...] + db_ref[...]
    dinv = lax.rsqrt(deg)[:N]
    y_ref[...] = x_ref[...] * dinv


def _make_y(x, deg_a, deg_b):
    return pl.pallas_call(
        _y_body,
        in_specs=[
            pl.BlockSpec((N, D), lambda: (0, 0)),
            pl.BlockSpec((NDEG, 1), lambda: (0, 0)),
            pl.BlockSpec((NDEG, 1), lambda: (0, 0)),
        ],
        out_specs=pl.BlockSpec((N, D), lambda: (0, 0)),
        out_shape=jax.ShapeDtypeStruct((N, D), jnp.float32),
    )(x, deg_a, deg_b)


def _ext_body(acc_a_ref, acc_b_ref, y_ref, da_ref, db_ref, bias_ref, lin_ref, ext_ref):
    deg = da_ref[...] + db_ref[...]
    dinv = lax.rsqrt(deg)[:N]
    acc = acc_a_ref[...] + acc_b_ref[...] + y_ref[...]
    emb = acc * dinv + bias_ref[...]
    linb = jnp.broadcast_to(lin_ref[...], (N, D))
    ext_ref[...] = jnp.concatenate([emb, linb], axis=1)


def _make_ext(acc_a, acc_b, y, deg_a, deg_b, bias, lin_table):
    return pl.pallas_call(
        _ext_body,
        in_specs=[
            pl.BlockSpec((N, D), lambda: (0, 0)),
            pl.BlockSpec((N, D), lambda: (0, 0)),
            pl.BlockSpec((N, D), lambda: (0, 0)),
            pl.BlockSpec((NDEG, 1), lambda: (0, 0)),
            pl.BlockSpec((NDEG, 1), lambda: (0, 0)),
            pl.BlockSpec((1, D), lambda: (0, 0)),
            pl.BlockSpec((N, 1), lambda: (0, 0)),
        ],
        out_specs=pl.BlockSpec((N, 2 * D), lambda: (0, 0)),
        out_shape=jax.ShapeDtypeStruct((N, 2 * D), jnp.float32),
    )(acc_a, acc_b, y, deg_a, deg_b, bias, lin_table)


def _fm_body(r_ref, lb_ref, o_ref):
    r = r_ref[...]
    a = r[:, 0:D]
    b = r[:, 2 * D:3 * D]
    fm = jnp.sum(a * b, axis=1, keepdims=True)
    lin = r[:, D:D + 1] + r[:, 3 * D:3 * D + 1]
    o_ref[...] = fm + lin + lb_ref[...]


def _fm(rows2, lin_bias):
    return pl.pallas_call(
        _fm_body,
        in_specs=[
            pl.BlockSpec((B, 4 * D), lambda: (0, 0)),
            pl.BlockSpec((1, 1), lambda: (0, 0)),
        ],
        out_specs=pl.BlockSpec((B, 1), lambda: (0, 0)),
        out_shape=jax.ShapeDtypeStruct((B, 1), jnp.float32),
    )(rows2, lin_bias)


def kernel(features, gcn_weight, gcn_bias, lin_table, lin_bias,
           interaction_pairs, edge_index):
    pad_len = EPAD - E
    pad_idx = (N + (jnp.arange(pad_len, dtype=jnp.int32) % NPAD))
    src_p = jnp.concatenate([edge_index[0], pad_idx]).reshape(NW, ECH, CHUNK)
    dst_p = jnp.concatenate([edge_index[1], pad_idx]).reshape(NW, ECH, CHUNK)
    pairs_flat = interaction_pairs.reshape(NW, PCH, CHUNK)

    ones128 = jnp.ones((CHUNK,), jnp.float32)
    zeros_deg = jnp.zeros((NDEG,), jnp.float32)
    zeros_acc = jnp.zeros((NT, D), jnp.float32)

    deg_parts = _deg_kernel(dst_p, ones128, zeros_deg)
    deg_a = deg_parts[0].reshape(NDEG, 1)
    deg_b = deg_parts[1].reshape(NDEG, 1)

    x = _xw(features, gcn_weight)

    y = _make_y(x, deg_a, deg_b)
    y_pad = jnp.pad(y, ((0, NPAD), (0, 0)))

    acc_parts = _edge_kernel(y_pad, src_p, dst_p, zeros_acc)

    ext = _make_ext(acc_parts[0, :N], acc_parts[1, :N], y, deg_a, deg_b,
                    gcn_bias.reshape(1, D), lin_table)

    rows = _pair_gather_kernel(ext, pairs_flat)

    out = _fm(rows.reshape(B, 4 * D), lin_bias.reshape(1, 1))
    return out.reshape(B)


# speedup vs baseline: 15.6418x; 15.6418x over previous
import functools

import jax
import jax.numpy as jnp
from jax import lax
from jax.experimental import pallas as pl
from jax.experimental.pallas import tpu as pltpu
from jax.experimental.pallas import tpu_sc as plsc

N = 10000
E = 320000
D = 16
B = 16384

NC = 2
NS = 16
NW = NC * NS
CHUNK = 128
W128 = 128

ECH = -(-E // (NW * CHUNK))
EW = ECH * CHUNK
EPAD = NW * EW

NPAD = 8
NT = N + NPAD
NDEG = 10240

PCH = (2 * B) // (NW * CHUNK)

_mesh = plsc.VectorSubcoreMesh(
    core_axis_name="c", subcore_axis_name="s", num_cores=NC, num_subcores=NS
)


@functools.partial(
    pl.kernel,
    out_type=jax.ShapeDtypeStruct((NC, NDEG), jnp.float32),
    mesh=_mesh,
    scratch_types=[
        pltpu.VMEM((ECH, CHUNK), jnp.int32),
        pltpu.VMEM((CHUNK,), jnp.float32),
        pltpu.VMEM_SHARED((NDEG,), jnp.float32),
    ],
)
def _deg_kernel(dst_hbm, ones_hbm, zeros_hbm, deg_out, dst_v, ones_v, shared_deg):
    cid = lax.axis_index("c")
    sid = lax.axis_index("s")
    wid = cid * NS + sid

    @pl.when(sid == 0)
    def _():
        pltpu.sync_copy(zeros_hbm, shared_deg)

    pltpu.sync_copy(ones_hbm, ones_v)
    pltpu.sync_copy(dst_hbm.at[wid], dst_v)
    plsc.subcore_barrier()

    @pl.loop(0, ECH)
    def _(k):
        pltpu.sync_copy(ones_v, shared_deg.at[dst_v.at[k]], add=True)

    plsc.subcore_barrier()

    @pl.when(sid == 0)
    def _():
        pltpu.sync_copy(shared_deg, deg_out.at[cid])


@functools.partial(
    pl.kernel,
    out_type=jax.ShapeDtypeStruct((NC, NT, W128), jnp.float32),
    mesh=_mesh,
    scratch_types=[
        pltpu.VMEM((ECH, CHUNK), jnp.int32),
        pltpu.VMEM((ECH, CHUNK), jnp.int32),
        pltpu.VMEM((CHUNK, W128), jnp.float32),
        pltpu.VMEM_SHARED((NT, W128), jnp.float32),
    ],
)
def _edge_kernel(y_hbm, src_hbm, dst_hbm, zeros_hbm, acc_out,
                 src_v, dst_v, rows_v, shared_acc):
    cid = lax.axis_index("c")
    sid = lax.axis_index("s")
    wid = cid * NS + sid

    @pl.when(sid == 0)
    def _():
        pltpu.sync_copy(zeros_hbm, shared_acc)

    pltpu.sync_copy(src_hbm.at[wid], src_v)
    pltpu.sync_copy(dst_hbm.at[wid], dst_v)
    plsc.subcore_barrier()

    @pl.loop(0, ECH)
    def _(k):
        pltpu.sync_copy(y_hbm.at[src_v.at[k]], rows_v)
        pltpu.sync_copy(rows_v, shared_acc.at[dst_v.at[k]], add=True)

    plsc.subcore_barrier()

    @pl.when(sid == 0)
    def _():
        pltpu.sync_copy(shared_acc, acc_out.at[cid])


@functools.partial(
    pl.kernel,
    out_type=jax.ShapeDtypeStruct((2 * B, W128), jnp.float32),
    mesh=_mesh,
    scratch_types=[
        pltpu.VMEM((PCH, CHUNK), jnp.int32),
        pltpu.VMEM((CHUNK, W128), jnp.float32),
    ],
)
def _pair_gather_kernel(ext_hbm, idx_hbm, rows_out, idx_v, rows_v):
    cid = lax.axis_index("c")
    sid = lax.axis_index("s")
    wid = cid * NS + sid

    pltpu.sync_copy(idx_hbm.at[wid], idx_v)

    @pl.loop(0, PCH)
    def _(k):
        pltpu.sync_copy(ext_hbm.at[idx_v.at[k]], rows_v)
        pltpu.sync_copy(
            rows_v, rows_out.at[pl.ds(wid * (PCH * CHUNK) + k * CHUNK, CHUNK)]
        )


def _mm_body(a_ref, w_ref, o_ref):
    o_ref[...] = jnp.dot(a_ref[...], w_ref[...], preferred_element_type=jnp.float32)


def _xw(features, gcn_weight):
    rows = 400
    return pl.pallas_call(
        _mm_body,
        grid=(N // rows,),
        in_specs=[
            pl.BlockSpec((rows, N), lambda i: (i, 0)),
            pl.BlockSpec((N, D), lambda i: (0, 0)),
        ],
        out_specs=pl.BlockSpec((rows, D), lambda i: (i, 0)),
        out_shape=jax.ShapeDtypeStruct((N, D), jnp.float32),
    )(features, gcn_weight)


def _y_body(x_ref, da_ref, db_ref, y_ref):
    deg = da_ref[...] + db_ref[...] + 1.0
    dinv = lax.rsqrt(deg)[:NT]
    y_ref[:, 0:D] = x_ref[...] * dinv
    y_ref[:, D:W128] = jnp.zeros((NT, W128 - D), jnp.float32)


def _make_y(x_pad, deg_a, deg_b):
    return pl.pallas_call(
        _y_body,
        in_specs=[
            pl.BlockSpec((NT, D), lambda: (0, 0)),
            pl.BlockSpec((NDEG, 1), lambda: (0, 0)),
            pl.BlockSpec((NDEG, 1), lambda: (0, 0)),
        ],
        out_specs=pl.BlockSpec((NT, W128), lambda: (0, 0)),
        out_shape=jax.ShapeDtypeStruct((NT, W128), jnp.float32),
    )(x_pad, deg_a, deg_b)


def _ext_body(acc_a_ref, acc_b_ref, y_ref, da_ref, db_ref, bias_ref, lin_ref,
              ext_ref):
    deg = da_ref[...] + db_ref[...] + 1.0
    dinv = lax.rsqrt(deg)[:N]
    acc = acc_a_ref[...] + acc_b_ref[...] + y_ref[...]
    emb = acc * dinv + bias_ref[...]
    ext_ref[:, 0:D] = emb
    ext_ref[:, D:2 * D] = jnp.broadcast_to(lin_ref[...], (N, D))
    ext_ref[:, 2 * D:W128] = jnp.zeros((N, W128 - 2 * D), jnp.float32)


def _make_ext(acc_a, acc_b, y, deg_a, deg_b, bias, lin_table):
    return pl.pallas_call(
        _ext_body,
        in_specs=[
            pl.BlockSpec((N, D), lambda: (0, 0)),
            pl.BlockSpec((N, D), lambda: (0, 0)),
            pl.BlockSpec((N, D), lambda: (0, 0)),
            pl.BlockSpec((NDEG, 1), lambda: (0, 0)),
            pl.BlockSpec((NDEG, 1), lambda: (0, 0)),
            pl.BlockSpec((1, D), lambda: (0, 0)),
            pl.BlockSpec((N, 1), lambda: (0, 0)),
        ],
        out_specs=pl.BlockSpec((N, W128), lambda: (0, 0)),
        out_shape=jax.ShapeDtypeStruct((N, W128), jnp.float32),
    )(acc_a, acc_b, y, deg_a, deg_b, bias, lin_table)


def _fm_body(r_ref, lb_ref, o_ref):
    r = r_ref[...]
    a = r[:, 0:D]
    b = r[:, W128:W128 + D]
    fm = jnp.sum(a * b, axis=1, keepdims=True)
    lin = r[:, D:D + 1] + r[:, W128 + D:W128 + D + 1]
    o_ref[...] = fm + lin + lb_ref[...]


def _fm(rows2, lin_bias):
    return pl.pallas_call(
        _fm_body,
        in_specs=[
            pl.BlockSpec((B, 2 * W128), lambda: (0, 0)),
            pl.BlockSpec((1, 1), lambda: (0, 0)),
        ],
        out_specs=pl.BlockSpec((B, 1), lambda: (0, 0)),
        out_shape=jax.ShapeDtypeStruct((B, 1), jnp.float32),
    )(rows2, lin_bias)


def kernel(features, gcn_weight, gcn_bias, lin_table, lin_bias,
           interaction_pairs, edge_index):
    pad_len = EPAD - E
    pad_idx = (N + (jnp.arange(pad_len, dtype=jnp.int32) % NPAD))
    src_p = jnp.concatenate([edge_index[0], pad_idx]).reshape(NW, ECH, CHUNK)
    dst_p = jnp.concatenate([edge_index[1], pad_idx]).reshape(NW, ECH, CHUNK)
    pairs_flat = interaction_pairs.reshape(NW, PCH, CHUNK)

    ones128 = jnp.ones((CHUNK,), jnp.float32)
    zeros_deg = jnp.zeros((NDEG,), jnp.float32)
    zeros_acc = jnp.zeros((NT, W128), jnp.float32)

    deg_parts = _deg_kernel(dst_p, ones128, zeros_deg)
    deg_a = deg_parts[0].reshape(NDEG, 1)
    deg_b = deg_parts[1].reshape(NDEG, 1)

    x = _xw(features, gcn_weight)
    x_pad = jnp.pad(x, ((0, NPAD), (0, 0)))

    y_wide = _make_y(x_pad, deg_a, deg_b)

    acc_parts = _edge_kernel(y_wide, src_p, dst_p, zeros_acc)

    ext = _make_ext(acc_parts[0, :N, 0:D], acc_parts[1, :N, 0:D],
                    y_wide[:N, 0:D], deg_a, deg_b,
                    gcn_bias.reshape(1, D), lin_table)

    rows = _pair_gather_kernel(ext, pairs_flat)

    out = _fm(rows.reshape(B, 2 * W128), lin_bias.reshape(1, 1))
    return out.reshape(B)


# speedup vs baseline: 17.1526x; 1.0966x over previous
import functools

import jax
import jax.numpy as jnp
from jax import lax
from jax.experimental import pallas as pl
from jax.experimental.pallas import tpu as pltpu
from jax.experimental.pallas import tpu_sc as plsc

N = 10000
E = 320000
D = 16
B = 16384

NC = 2
NS = 16
NW = NC * NS
CHUNK = 128
W128 = 128

NCH = E // CHUNK
WCH = NCH // NW
XCH = NCH - NW * WCH
SLAB = 88
XROW = SLAB
VROWS = 96
ASTART_MAX = ((NCH - SLAB) // 8) * 8

NDEG = 10240

PCH = B // (NW * CHUNK)

MMROWS = 400

_mesh = plsc.VectorSubcoreMesh(
    core_axis_name="c", subcore_axis_name="s", num_cores=NC, num_subcores=NS
)


def _slab_base(wid):
    start = wid * WCH
    astart = jnp.minimum((start // 8) * 8, ASTART_MAX)
    return astart, start - astart


_DEG_KW = dict(
    out_type=jax.ShapeDtypeStruct((NC, NDEG), jnp.float32),
    mesh=_mesh,
    scratch_types=[
        pltpu.VMEM((VROWS, CHUNK), jnp.int32),
        pltpu.VMEM((CHUNK,), jnp.float32),
        pltpu.VMEM_SHARED((NDEG,), jnp.float32),
    ],
)


def _deg_body(dst_hbm, ones_hbm, zeros_hbm, deg_out, dst_v, ones_v, shared_deg):
    cid = lax.axis_index("c")
    sid = lax.axis_index("s")
    wid = cid * NS + sid
    astart, base = _slab_base(wid)

    @pl.when(sid == 0)
    def _():
        pltpu.sync_copy(zeros_hbm, shared_deg)

    pltpu.sync_copy(ones_hbm, ones_v)
    pltpu.sync_copy(dst_hbm.at[pl.ds(astart, SLAB)], dst_v.at[pl.ds(0, SLAB)])

    @pl.when(wid < XCH)
    def _():
        pltpu.sync_copy(dst_hbm.at[pl.ds(NW * WCH, XCH)],
                        dst_v.at[pl.ds(XROW, XCH)])

    plsc.subcore_barrier()

    @pl.loop(0, WCH)
    def _(k):
        pltpu.sync_copy(ones_v, shared_deg.at[dst_v.at[base + k]], add=True)

    @pl.when(wid < XCH)
    def _():
        pltpu.sync_copy(ones_v, shared_deg.at[dst_v.at[XROW + wid]], add=True)

    plsc.subcore_barrier()

    @pl.when(sid == 0)
    def _():
        pltpu.sync_copy(shared_deg, deg_out.at[cid])


_EDGE_KW = dict(
    out_type=jax.ShapeDtypeStruct((NC, N, W128), jnp.float32),
    mesh=_mesh,
    scratch_types=[
        pltpu.VMEM((VROWS, CHUNK), jnp.int32),
        pltpu.VMEM((VROWS, CHUNK), jnp.int32),
        pltpu.VMEM((CHUNK, W128), jnp.float32),
        pltpu.VMEM_SHARED((N, W128), jnp.float32),
    ],
)


def _edge_body(y_hbm, src_hbm, dst_hbm, zeros_hbm, acc_out,
               src_v, dst_v, r0, shared_acc):
    cid = lax.axis_index("c")
    sid = lax.axis_index("s")
    wid = cid * NS + sid
    astart, base = _slab_base(wid)

    @pl.when(sid == 0)
    def _():
        pltpu.sync_copy(zeros_hbm, shared_acc)

    pltpu.sync_copy(src_hbm.at[pl.ds(astart, SLAB)], src_v.at[pl.ds(0, SLAB)])
    pltpu.sync_copy(dst_hbm.at[pl.ds(astart, SLAB)], dst_v.at[pl.ds(0, SLAB)])

    @pl.when(wid < XCH)
    def _():
        pltpu.sync_copy(src_hbm.at[pl.ds(NW * WCH, XCH)],
                        src_v.at[pl.ds(XROW, XCH)])
        pltpu.sync_copy(dst_hbm.at[pl.ds(NW * WCH, XCH)],
                        dst_v.at[pl.ds(XROW, XCH)])

    plsc.subcore_barrier()

    @pl.loop(0, WCH)
    def _(k):
        c = base + k
        pltpu.sync_copy(y_hbm.at[src_v.at[c]], r0)
        pltpu.sync_copy(r0, shared_acc.at[dst_v.at[c]], add=True)

    @pl.when(wid < XCH)
    def _():
        pltpu.sync_copy(y_hbm.at[src_v.at[XROW + wid]], r0)
        pltpu.sync_copy(r0, shared_acc.at[dst_v.at[XROW + wid]], add=True)

    plsc.subcore_barrier()

    @pl.when(sid == 0)
    def _():
        pltpu.sync_copy(shared_acc, acc_out.at[cid])


_PAIR_KW = dict(
    out_type=(jax.ShapeDtypeStruct((B, W128), jnp.float32),
              jax.ShapeDtypeStruct((B, W128), jnp.float32)),
    mesh=_mesh,
    scratch_types=[
        pltpu.VMEM((PCH, CHUNK), jnp.int32),
        pltpu.VMEM((PCH, CHUNK), jnp.int32),
        pltpu.VMEM((CHUNK, W128), jnp.float32),
        pltpu.VMEM_SHARED((N, W128), jnp.float32),
    ],
)


def _pair_gather_body(ext_hbm, ia_hbm, ib_hbm, out_a, out_b,
                      ia_v, ib_v, rows_v, shared_ext):
    cid = lax.axis_index("c")
    sid = lax.axis_index("s")
    wid = cid * NS + sid

    @pl.when(sid == 0)
    def _():
        pltpu.sync_copy(ext_hbm, shared_ext)

    pltpu.sync_copy(ia_hbm.at[wid], ia_v)
    pltpu.sync_copy(ib_hbm.at[wid], ib_v)
    plsc.subcore_barrier()

    @pl.loop(0, PCH)
    def _(k):
        pltpu.sync_copy(shared_ext.at[ia_v.at[k]], rows_v)
        pltpu.sync_copy(rows_v,
                        out_a.at[pl.ds(wid * (PCH * CHUNK) + k * CHUNK, CHUNK)])

    @pl.loop(0, PCH)
    def _(k):
        pltpu.sync_copy(shared_ext.at[ib_v.at[k]], rows_v)
        pltpu.sync_copy(rows_v,
                        out_b.at[pl.ds(wid * (PCH * CHUNK) + k * CHUNK, CHUNK)])


_deg_kernel = pl.kernel(_deg_body, **_DEG_KW)
_edge_kernel = pl.kernel(_edge_body, **_EDGE_KW)
_pair_gather_kernel = pl.kernel(_pair_gather_body, **_PAIR_KW)


def _mmy_body(a_ref, w_ref, dgt_ref, y_ref):
    x = jnp.dot(a_ref[...], w_ref[...], preferred_element_type=jnp.float32)
    deg = dgt_ref[:, 0:1] + dgt_ref[:, 1:2] + 1.0
    y_ref[:, 0:D] = x * lax.rsqrt(deg)
    y_ref[:, D:W128] = jnp.zeros((MMROWS, W128 - D), jnp.float32)


def _xw_y(features, gcn_weight, deg_t):
    return pl.pallas_call(
        _mmy_body,
        grid=(N // MMROWS,),
        in_specs=[
            pl.BlockSpec((MMROWS, N), lambda i: (i, 0)),
            pl.BlockSpec((N, D), lambda i: (0, 0)),
            pl.BlockSpec((MMROWS, NC), lambda i: (i, 0)),
        ],
        out_specs=pl.BlockSpec((MMROWS, W128), lambda i: (i, 0)),
        out_shape=jax.ShapeDtypeStruct((N, W128), jnp.float32),
    )(features, gcn_weight, deg_t)


def _ext_body(acc_ref, y_ref, dgt_ref, bias_ref, lin_ref, ext_ref):
    deg = dgt_ref[:, 0:1] + dgt_ref[:, 1:2] + 1.0
    dinv = lax.rsqrt(deg)
    acc = acc_ref[0, :, 0:D] + acc_ref[1, :, 0:D] + y_ref[:, 0:D]
    ext_ref[:, 0:D] = acc * dinv + bias_ref[...]
    ext_ref[:, D:2 * D] = jnp.broadcast_to(lin_ref[...], (N, D))
    ext_ref[:, 2 * D:W128] = jnp.zeros((N, W128 - 2 * D), jnp.float32)


def _make_ext(acc_parts, y, deg_t, bias, lin_table):
    return pl.pallas_call(
        _ext_body,
        in_specs=[
            pl.BlockSpec((NC, N, W128), lambda: (0, 0, 0)),
            pl.BlockSpec((N, W128), lambda: (0, 0)),
            pl.BlockSpec((N, NC), lambda: (0, 0)),
            pl.BlockSpec((1, D), lambda: (0, 0)),
            pl.BlockSpec((N, 1), lambda: (0, 0)),
        ],
        out_specs=pl.BlockSpec((N, W128), lambda: (0, 0)),
        out_shape=jax.ShapeDtypeStruct((N, W128), jnp.float32),
    )(acc_parts, y, deg_t, bias, lin_table)


def _fm_body(a_ref, b_ref, lb_ref, o_ref):
    a = a_ref[:, 0:D]
    b = b_ref[:, 0:D]
    fm = jnp.sum(a * b, axis=1, keepdims=True)
    lin = a_ref[:, D:D + 1] + b_ref[:, D:D + 1]
    o_ref[...] = fm + lin + lb_ref[...]


def _fm(rows_a, rows_b, lin_bias):
    return pl.pallas_call(
        _fm_body,
        in_specs=[
            pl.BlockSpec((B, W128), lambda: (0, 0)),
            pl.BlockSpec((B, W128), lambda: (0, 0)),
            pl.BlockSpec((1, 1), lambda: (0, 0)),
        ],
        out_specs=pl.BlockSpec((B, 1), lambda: (0, 0)),
        out_shape=jax.ShapeDtypeStruct((B, 1), jnp.float32),
    )(rows_a, rows_b, lin_bias)


def kernel(features, gcn_weight, gcn_bias, lin_table, lin_bias,
           interaction_pairs, edge_index):
    src_c = edge_index[0].reshape(NCH, CHUNK)
    dst_c = edge_index[1].reshape(NCH, CHUNK)
    idx_a = interaction_pairs[:, 0].reshape(NW, PCH, CHUNK)
    idx_b = interaction_pairs[:, 1].reshape(NW, PCH, CHUNK)

    ones128 = jnp.ones((CHUNK,), jnp.float32)
    zeros_deg = jnp.zeros((NDEG,), jnp.float32)
    zeros_acc = jnp.zeros((N, W128), jnp.float32)

    deg_parts = _deg_kernel(dst_c, ones128, zeros_deg)
    deg_t = deg_parts.T

    y = _xw_y(features, gcn_weight, deg_t)

    acc_parts = _edge_kernel(y, src_c, dst_c, zeros_acc)

    ext = _make_ext(acc_parts, y, deg_t[:N], gcn_bias.reshape(1, D), lin_table)

    rows_a, rows_b = _pair_gather_kernel(ext, idx_a, idx_b)

    out = _fm(rows_a, rows_b, lin_bias.reshape(1, 1))
    return out.reshape(B)


# speedup vs baseline: 18.0322x; 1.0513x over previous
import functools

import jax
import jax.numpy as jnp
from jax import lax
from jax.experimental import pallas as pl
from jax.experimental.pallas import tpu as pltpu
from jax.experimental.pallas import tpu_sc as plsc

N = 10000
E = 320000
D = 16
B = 16384

NC = 2
NS = 16
NW = NC * NS
CHUNK = 128
W128 = 128

NCH = E // CHUNK
WCH = NCH // NW
XCH = NCH - NW * WCH
SLAB = 88
XROW = SLAB
VROWS = 96
ASTART_MAX = ((NCH - SLAB) // 8) * 8

NDEG = 10240

PCH = B // (NW * CHUNK)

MMROWS = 400

_mesh = plsc.VectorSubcoreMesh(
    core_axis_name="c", subcore_axis_name="s", num_cores=NC, num_subcores=NS
)


def _slab_base(wid):
    start = wid * WCH
    astart = jnp.minimum((start // 8) * 8, ASTART_MAX)
    return astart, start - astart


_DEG_KW = dict(
    out_type=jax.ShapeDtypeStruct((NC, NDEG), jnp.float32),
    mesh=_mesh,
    scratch_types=[
        pltpu.VMEM((VROWS, CHUNK), jnp.int32),
        pltpu.VMEM((CHUNK,), jnp.float32),
        pltpu.VMEM_SHARED((NDEG,), jnp.float32),
    ],
)


def _deg_body(edges_hbm, ones_hbm, zeros_hbm, deg_out, dst_v, ones_v, shared_deg):
    cid = lax.axis_index("c")
    sid = lax.axis_index("s")
    wid = cid * NS + sid
    astart, base = _slab_base(wid)

    @pl.when(sid == 0)
    def _():
        pltpu.sync_copy(zeros_hbm, shared_deg)

    pltpu.sync_copy(ones_hbm, ones_v)
    pltpu.sync_copy(edges_hbm.at[1, pl.ds(astart, SLAB)],
                    dst_v.at[pl.ds(0, SLAB)])

    @pl.when(wid < XCH)
    def _():
        pltpu.sync_copy(edges_hbm.at[1, pl.ds(NW * WCH, XCH)],
                        dst_v.at[pl.ds(XROW, XCH)])

    plsc.subcore_barrier()

    @pl.loop(0, WCH)
    def _(k):
        pltpu.sync_copy(ones_v, shared_deg.at[dst_v.at[base + k]], add=True)

    @pl.when(wid < XCH)
    def _():
        pltpu.sync_copy(ones_v, shared_deg.at[dst_v.at[XROW + wid]], add=True)

    plsc.subcore_barrier()

    @pl.when(sid == 0)
    def _():
        pltpu.sync_copy(shared_deg, deg_out.at[cid])


_EDGE_KW = dict(
    out_type=jax.ShapeDtypeStruct((NC, N, W128), jnp.float32),
    mesh=_mesh,
    scratch_types=[
        pltpu.VMEM((VROWS, CHUNK), jnp.int32),
        pltpu.VMEM((VROWS, CHUNK), jnp.int32),
        pltpu.VMEM((CHUNK, W128), jnp.float32),
        pltpu.VMEM_SHARED((N, W128), jnp.float32),
    ],
)


def _edge_body(y_hbm, edges_hbm, zeros_hbm, acc_out,
               src_v, dst_v, r0, shared_acc):
    cid = lax.axis_index("c")
    sid = lax.axis_index("s")
    wid = cid * NS + sid
    astart, base = _slab_base(wid)

    @pl.when(sid == 0)
    def _():
        pltpu.sync_copy(zeros_hbm, shared_acc)

    pltpu.sync_copy(edges_hbm.at[0, pl.ds(astart, SLAB)],
                    src_v.at[pl.ds(0, SLAB)])
    pltpu.sync_copy(edges_hbm.at[1, pl.ds(astart, SLAB)],
                    dst_v.at[pl.ds(0, SLAB)])

    @pl.when(wid < XCH)
    def _():
        pltpu.sync_copy(edges_hbm.at[0, pl.ds(NW * WCH, XCH)],
                        src_v.at[pl.ds(XROW, XCH)])
        pltpu.sync_copy(edges_hbm.at[1, pl.ds(NW * WCH, XCH)],
                        dst_v.at[pl.ds(XROW, XCH)])

    plsc.subcore_barrier()

    @pl.loop(0, WCH)
    def _(k):
        c = base + k
        pltpu.sync_copy(y_hbm.at[src_v.at[c]], r0)
        pltpu.sync_copy(r0, shared_acc.at[dst_v.at[c]], add=True)

    @pl.when(wid < XCH)
    def _():
        pltpu.sync_copy(y_hbm.at[src_v.at[XROW + wid]], r0)
        pltpu.sync_copy(r0, shared_acc.at[dst_v.at[XROW + wid]], add=True)

    plsc.subcore_barrier()

    @pl.when(sid == 0)
    def _():
        pltpu.sync_copy(shared_acc, acc_out.at[cid])


STG = 640

_PAIR_KW = dict(
    out_type=(jax.ShapeDtypeStruct((B, W128), jnp.float32),
              jax.ShapeDtypeStruct((B, W128), jnp.float32)),
    mesh=_mesh,
    scratch_types=[
        pltpu.VMEM((PCH, CHUNK), jnp.int32),
        pltpu.VMEM((PCH, CHUNK), jnp.int32),
        pltpu.VMEM((CHUNK, W128), jnp.float32),
        pltpu.VMEM((CHUNK, W128), jnp.float32),
        pltpu.VMEM_SHARED((N, W128), jnp.float32),
        pltpu.SemaphoreType.DMA,
        pltpu.SemaphoreType.DMA,
    ],
)


def _pair_gather_body(ext_hbm, ia_hbm, ib_hbm, out_a, out_b,
                      ia_v, ib_v, r0, r1, shared_ext, w0, w1):
    cid = lax.axis_index("c")
    sid = lax.axis_index("s")
    wid = cid * NS + sid

    @pl.when(sid < NS - 1)
    def _():
        pltpu.sync_copy(ext_hbm.at[pl.ds(sid * STG, STG)],
                        shared_ext.at[pl.ds(sid * STG, STG)])

    @pl.when(sid == NS - 1)
    def _():
        pltpu.sync_copy(ext_hbm.at[pl.ds((NS - 1) * STG, N - (NS - 1) * STG)],
                        shared_ext.at[pl.ds((NS - 1) * STG, N - (NS - 1) * STG)])

    pltpu.sync_copy(ia_hbm.at[wid], ia_v)
    pltpu.sync_copy(ib_hbm.at[wid], ib_v)
    plsc.subcore_barrier()

    work = [(ia_v, out_a, k) for k in range(PCH)] +            [(ib_v, out_b, k) for k in range(PCH)]
    bufs = (r0, r1)
    sems = (w0, w1)
    for t, (idx_v, out, k) in enumerate(work):
        r, w = bufs[t % 2], sems[t % 2]
        if t >= 2:
            p_idx, p_out, p_k = work[t - 2]
            pltpu.make_async_copy(
                r, p_out.at[pl.ds(wid * (PCH * CHUNK) + p_k * CHUNK, CHUNK)],
                w).wait()
        pltpu.sync_copy(shared_ext.at[idx_v.at[k]], r)
        pltpu.async_copy(
            r, out.at[pl.ds(wid * (PCH * CHUNK) + k * CHUNK, CHUNK)], w)
    for t in (len(work) - 2, len(work) - 1):
        idx_v, out, k = work[t]
        pltpu.make_async_copy(
            bufs[t % 2],
            out.at[pl.ds(wid * (PCH * CHUNK) + k * CHUNK, CHUNK)],
            sems[t % 2]).wait()


_deg_kernel = pl.kernel(_deg_body, **_DEG_KW)
_edge_kernel = pl.kernel(_edge_body, **_EDGE_KW)
_pair_gather_kernel = pl.kernel(_pair_gather_body, **_PAIR_KW)


def _mm_body(a_ref, w_ref, x_ref):
    x_ref[...] = jnp.dot(a_ref[...], w_ref[...],
                         preferred_element_type=jnp.float32)


def _xw(features, gcn_weight):
    return pl.pallas_call(
        _mm_body,
        grid=(N // MMROWS,),
        in_specs=[
            pl.BlockSpec((MMROWS, N), lambda i: (i, 0)),
            pl.BlockSpec((N, D), lambda i: (0, 0)),
        ],
        out_specs=pl.BlockSpec((MMROWS, D), lambda i: (i, 0)),
        out_shape=jax.ShapeDtypeStruct((N, D), jnp.float32),
    )(features, gcn_weight)


def _y_body(x_ref, dgt_ref, y_ref):
    deg = dgt_ref[:, 0:1] + dgt_ref[:, 1:2] + 1.0
    y_ref[:, 0:D] = x_ref[...] * lax.rsqrt(deg)
    y_ref[:, D:W128] = jnp.zeros((N, W128 - D), jnp.float32)


def _make_y(x, deg_t):
    return pl.pallas_call(
        _y_body,
        in_specs=[
            pl.BlockSpec((N, D), lambda: (0, 0)),
            pl.BlockSpec((N, NC), lambda: (0, 0)),
        ],
        out_specs=pl.BlockSpec((N, W128), lambda: (0, 0)),
        out_shape=jax.ShapeDtypeStruct((N, W128), jnp.float32),
    )(x, deg_t)


def _ext_body(acc_ref, y_ref, dgt_ref, bias_ref, lin_ref, ext_ref):
    deg = dgt_ref[:, 0:1] + dgt_ref[:, 1:2] + 1.0
    dinv = lax.rsqrt(deg)
    acc = acc_ref[0, :, 0:D] + acc_ref[1, :, 0:D] + y_ref[:, 0:D]
    ext_ref[:, 0:D] = acc * dinv + bias_ref[...]
    ext_ref[:, D:2 * D] = jnp.broadcast_to(lin_ref[...], (N, D))
    ext_ref[:, 2 * D:W128] = jnp.zeros((N, W128 - 2 * D), jnp.float32)


def _make_ext(acc_parts, y, deg_t, bias, lin_table):
    return pl.pallas_call(
        _ext_body,
        in_specs=[
            pl.BlockSpec((NC, N, W128), lambda: (0, 0, 0)),
            pl.BlockSpec((N, W128), lambda: (0, 0)),
            pl.BlockSpec((N, NC), lambda: (0, 0)),
            pl.BlockSpec((1, D), lambda: (0, 0)),
            pl.BlockSpec((N, 1), lambda: (0, 0)),
        ],
        out_specs=pl.BlockSpec((N, W128), lambda: (0, 0)),
        out_shape=jax.ShapeDtypeStruct((N, W128), jnp.float32),
    )(acc_parts, y, deg_t, bias, lin_table)


def _fm_body(a_ref, b_ref, lb_ref, o_ref):
    a = a_ref[:, 0:D]
    b = b_ref[:, 0:D]
    fm = jnp.sum(a * b, axis=1, keepdims=True)
    lin = a_ref[:, D:D + 1] + b_ref[:, D:D + 1]
    o_ref[...] = fm + lin + lb_ref[...]


def _fm(rows_a, rows_b, lin_bias):
    return pl.pallas_call(
        _fm_body,
        in_specs=[
            pl.BlockSpec((B, W128), lambda: (0, 0)),
            pl.BlockSpec((B, W128), lambda: (0, 0)),
            pl.BlockSpec((1, 1), lambda: (0, 0)),
        ],
        out_specs=pl.BlockSpec((B, 1), lambda: (0, 0)),
        out_shape=jax.ShapeDtypeStruct((B, 1), jnp.float32),
    )(rows_a, rows_b, lin_bias)


def kernel(features, gcn_weight, gcn_bias, lin_table, lin_bias,
           interaction_pairs, edge_index):
    edges3 = edge_index.reshape(2, NCH, CHUNK)
    idx_a = interaction_pairs[:, 0].reshape(NW, PCH, CHUNK)
    idx_b = interaction_pairs[:, 1].reshape(NW, PCH, CHUNK)

    ones128 = jnp.ones((CHUNK,), jnp.float32)
    zeros_deg = jnp.zeros((NDEG,), jnp.float32)
    zeros_acc = jnp.zeros((N, W128), jnp.float32)

    deg_parts = _deg_kernel(edges3, ones128, zeros_deg)
    deg_t = deg_parts.T

    x = _xw(features, gcn_weight)
    y = _make_y(x, deg_t[:N])

    acc_parts = _edge_kernel(y, edges3, zeros_acc)

    ext = _make_ext(acc_parts, y, deg_t[:N], gcn_bias.reshape(1, D), lin_table)

    rows_a, rows_b = _pair_gather_kernel(ext, idx_a, idx_b)

    out = _fm(rows_a, rows_b, lin_bias.reshape(1, 1))
    return out.reshape(B)


# speedup vs baseline: 19.2017x; 1.0649x over previous
import functools

import jax
import jax.numpy as jnp
from jax import lax
from jax.experimental import pallas as pl
from jax.experimental.pallas import tpu as pltpu
from jax.experimental.pallas import tpu_sc as plsc

N = 10000
E = 320000
D = 16
B = 16384

NC = 2
NS = 16
NW = NC * NS
CHUNK = 128
W128 = 128

NCH = E // CHUNK
WCH = NCH // NW
XCH = NCH - NW * WCH
SLAB = 88
XROW = SLAB
VROWS = 96
ASTART_MAX = ((NCH - SLAB) // 8) * 8

NDEG = 10240

PCH = B // (NW * CHUNK)

MMROWS = 400

_mesh = plsc.VectorSubcoreMesh(
    core_axis_name="c", subcore_axis_name="s", num_cores=NC, num_subcores=NS
)


def _slab_base(wid):
    start = wid * WCH
    astart = jnp.minimum((start // 8) * 8, ASTART_MAX)
    return astart, start - astart


_DEG_KW = dict(
    out_type=jax.ShapeDtypeStruct((NC, NDEG), jnp.float32),
    mesh=_mesh,
    scratch_types=[
        pltpu.VMEM((VROWS, CHUNK), jnp.int32),
        pltpu.VMEM((CHUNK,), jnp.float32),
        pltpu.VMEM_SHARED((NDEG,), jnp.float32),
    ],
)


def _deg_body(edges_hbm, ones_hbm, zeros_hbm, deg_out, dst_v, ones_v, shared_deg):
    cid = lax.axis_index("c")
    sid = lax.axis_index("s")
    wid = cid * NS + sid
    astart, base = _slab_base(wid)

    @pl.when(sid == 0)
    def _():
        pltpu.sync_copy(zeros_hbm, shared_deg)

    pltpu.sync_copy(ones_hbm, ones_v)
    pltpu.sync_copy(edges_hbm.at[1, pl.ds(astart, SLAB)],
                    dst_v.at[pl.ds(0, SLAB)])

    @pl.when(wid < XCH)
    def _():
        pltpu.sync_copy(edges_hbm.at[1, pl.ds(NW * WCH, XCH)],
                        dst_v.at[pl.ds(XROW, XCH)])

    plsc.subcore_barrier()

    @pl.loop(0, WCH)
    def _(k):
        pltpu.sync_copy(ones_v, shared_deg.at[dst_v.at[base + k]], add=True)

    @pl.when(wid < XCH)
    def _():
        pltpu.sync_copy(ones_v, shared_deg.at[dst_v.at[XROW + wid]], add=True)

    plsc.subcore_barrier()

    @pl.when(sid == 0)
    def _():
        pltpu.sync_copy(shared_deg, deg_out.at[cid])


HALF0 = 40
HALF1 = WCH - HALF0
SLAB2 = 48
XROW2 = SLAB2
VROWS2 = 56
ASTART2_MAX = ((NCH - SLAB2) // 8) * 8

_EDGE_KW = dict(
    out_type=jax.ShapeDtypeStruct((NC, N, W128), jnp.float32),
    mesh=_mesh,
    scratch_types=[
        pltpu.VMEM((VROWS2, CHUNK), jnp.int32),
        pltpu.VMEM((VROWS2, CHUNK), jnp.int32),
        pltpu.VMEM((CHUNK, W128), jnp.float32),
        pltpu.VMEM((CHUNK, W128), jnp.float32),
        pltpu.VMEM_SHARED((N, W128), jnp.float32),
        pltpu.SemaphoreType.DMA,
        pltpu.SemaphoreType.DMA,
    ],
)


def _make_edge_body(off, cnt, with_extras):
    def body(y_hbm, edges_hbm, init_hbm, acc_out,
             src_v, dst_v, r0, r1, shared_acc, g0, g1):
        cid = lax.axis_index("c")
        sid = lax.axis_index("s")
        wid = cid * NS + sid
        start = wid * WCH + off
        astart = jnp.minimum((start // 8) * 8, ASTART2_MAX)
        base = start - astart

        @pl.when(sid == 0)
        def _():
            pltpu.sync_copy(init_hbm.at[cid], shared_acc)

        pltpu.sync_copy(edges_hbm.at[0, pl.ds(astart, SLAB2)],
                        src_v.at[pl.ds(0, SLAB2)])
        pltpu.sync_copy(edges_hbm.at[1, pl.ds(astart, SLAB2)],
                        dst_v.at[pl.ds(0, SLAB2)])

        if with_extras:
            @pl.when(wid < XCH)
            def _():
                pltpu.sync_copy(edges_hbm.at[0, pl.ds(NW * WCH, XCH)],
                                src_v.at[pl.ds(XROW2, XCH)])
                pltpu.sync_copy(edges_hbm.at[1, pl.ds(NW * WCH, XCH)],
                                dst_v.at[pl.ds(XROW2, XCH)])

        plsc.subcore_barrier()

        pltpu.async_copy(y_hbm.at[src_v.at[base]], r0, g0)

        @pl.loop(0, cnt // 2)
        def _(j):
            c0 = base + 2 * j
            c1 = base + 2 * j + 1
            pltpu.make_async_copy(y_hbm.at[src_v.at[c0]], r0, g0).wait()
            pltpu.async_copy(y_hbm.at[src_v.at[c1]], r1, g1)
            pltpu.sync_copy(r0, shared_acc.at[dst_v.at[c0]], add=True)
            pltpu.make_async_copy(y_hbm.at[src_v.at[c1]], r1, g1).wait()

            @pl.when(j < cnt // 2 - 1)
            def _():
                pltpu.async_copy(y_hbm.at[src_v.at[c0 + 2]], r0, g0)

            pltpu.sync_copy(r1, shared_acc.at[dst_v.at[c1]], add=True)

        if with_extras:
            @pl.when(wid < XCH)
            def _():
                pltpu.sync_copy(y_hbm.at[src_v.at[XROW2 + wid]], r0)
                pltpu.sync_copy(r0, shared_acc.at[dst_v.at[XROW2 + wid]],
                                add=True)

        plsc.subcore_barrier()

        @pl.when(sid == 0)
        def _():
            pltpu.sync_copy(shared_acc, acc_out.at[cid])

    return body


_edge_body0 = _make_edge_body(0, HALF0, False)
_edge_body1 = _make_edge_body(HALF0, HALF1, True)


STG = 640

_PAIR_KW = dict(
    out_type=(jax.ShapeDtypeStruct((B, W128), jnp.float32),
              jax.ShapeDtypeStruct((B, W128), jnp.float32)),
    mesh=_mesh,
    scratch_types=[
        pltpu.VMEM((PCH, CHUNK), jnp.int32),
        pltpu.VMEM((PCH, CHUNK), jnp.int32),
        pltpu.VMEM((CHUNK, W128), jnp.float32),
        pltpu.VMEM((CHUNK, W128), jnp.float32),
        pltpu.VMEM_SHARED((N, W128), jnp.float32),
        pltpu.SemaphoreType.DMA,
        pltpu.SemaphoreType.DMA,
    ],
)


def _pair_gather_body(ext_hbm, ia_hbm, ib_hbm, out_a, out_b,
                      ia_v, ib_v, r0, r1, shared_ext, w0, w1):
    cid = lax.axis_index("c")
    sid = lax.axis_index("s")
    wid = cid * NS + sid

    @pl.when(sid < NS - 1)
    def _():
        pltpu.sync_copy(ext_hbm.at[pl.ds(sid * STG, STG)],
                        shared_ext.at[pl.ds(sid * STG, STG)])

    @pl.when(sid == NS - 1)
    def _():
        pltpu.sync_copy(ext_hbm.at[pl.ds((NS - 1) * STG, N - (NS - 1) * STG)],
                        shared_ext.at[pl.ds((NS - 1) * STG, N - (NS - 1) * STG)])

    pltpu.sync_copy(ia_hbm.at[wid], ia_v)
    pltpu.sync_copy(ib_hbm.at[wid], ib_v)
    plsc.subcore_barrier()

    work = [(ia_v, out_a, k) for k in range(PCH)] +            [(ib_v, out_b, k) for k in range(PCH)]
    bufs = (r0, r1)
    sems = (w0, w1)
    for t, (idx_v, out, k) in enumerate(work):
        r, w = bufs[t % 2], sems[t % 2]
        if t >= 2:
            p_idx, p_out, p_k = work[t - 2]
            pltpu.make_async_copy(
                r, p_out.at[pl.ds(wid * (PCH * CHUNK) + p_k * CHUNK, CHUNK)],
                w).wait()
        pltpu.sync_copy(shared_ext.at[idx_v.at[k]], r)
        pltpu.async_copy(
            r, out.at[pl.ds(wid * (PCH * CHUNK) + k * CHUNK, CHUNK)], w)
    for t in (len(work) - 2, len(work) - 1):
        idx_v, out, k = work[t]
        pltpu.make_async_copy(
            bufs[t % 2],
            out.at[pl.ds(wid * (PCH * CHUNK) + k * CHUNK, CHUNK)],
            sems[t % 2]).wait()


_deg_kernel = pl.kernel(_deg_body, **_DEG_KW)
_edge_kernel0 = pl.kernel(_edge_body0, **_EDGE_KW)
_edge_kernel1 = pl.kernel(_edge_body1, **_EDGE_KW)
_pair_gather_kernel = pl.kernel(_pair_gather_body, **_PAIR_KW)


def _mm_body(a_ref, w_ref, x_ref):
    x_ref[...] = jnp.dot(a_ref[...], w_ref[...],
                         preferred_element_type=jnp.float32)


def _xw(features, gcn_weight):
    return pl.pallas_call(
        _mm_body,
        grid=(N // MMROWS,),
        in_specs=[
            pl.BlockSpec((MMROWS, N), lambda i: (i, 0)),
            pl.BlockSpec((N, D), lambda i: (0, 0)),
        ],
        out_specs=pl.BlockSpec((MMROWS, D), lambda i: (i, 0)),
        out_shape=jax.ShapeDtypeStruct((N, D), jnp.float32),
    )(features, gcn_weight)


def _y_body(x_ref, dgt_ref, y_ref):
    deg = dgt_ref[:, 0:1] + dgt_ref[:, 1:2] + 1.0
    y_ref[:, 0:D] = x_ref[...] * lax.rsqrt(deg)
    y_ref[:, D:W128] = jnp.zeros((N, W128 - D), jnp.float32)


def _make_y(x, deg_t):
    return pl.pallas_call(
        _y_body,
        in_specs=[
            pl.BlockSpec((N, D), lambda: (0, 0)),
            pl.BlockSpec((N, NC), lambda: (0, 0)),
        ],
        out_specs=pl.BlockSpec((N, W128), lambda: (0, 0)),
        out_shape=jax.ShapeDtypeStruct((N, W128), jnp.float32),
    )(x, deg_t)


def _ext_body(acc_ref, y_ref, dgt_ref, bias_ref, lin_ref, ext_ref):
    deg = dgt_ref[:, 0:1] + dgt_ref[:, 1:2] + 1.0
    dinv = lax.rsqrt(deg)
    acc = acc_ref[0, :, 0:D] + acc_ref[1, :, 0:D] + y_ref[:, 0:D]
    ext_ref[:, 0:D] = acc * dinv + bias_ref[...]
    ext_ref[:, D:2 * D] = jnp.broadcast_to(lin_ref[...], (N, D))
    ext_ref[:, 2 * D:W128] = jnp.zeros((N, W128 - 2 * D), jnp.float32)


def _make_ext(acc_parts, y, deg_t, bias, lin_table):
    return pl.pallas_call(
        _ext_body,
        in_specs=[
            pl.BlockSpec((NC, N, W128), lambda: (0, 0, 0)),
            pl.BlockSpec((N, W128), lambda: (0, 0)),
            pl.BlockSpec((N, NC), lambda: (0, 0)),
            pl.BlockSpec((1, D), lambda: (0, 0)),
            pl.BlockSpec((N, 1), lambda: (0, 0)),
        ],
        out_specs=pl.BlockSpec((N, W128), lambda: (0, 0)),
        out_shape=jax.ShapeDtypeStruct((N, W128), jnp.float32),
    )(acc_parts, y, deg_t, bias, lin_table)


def _fm_body(a_ref, b_ref, lb_ref, o_ref):
    a = a_ref[:, 0:D]
    b = b_ref[:, 0:D]
    fm = jnp.sum(a * b, axis=1, keepdims=True)
    lin = a_ref[:, D:D + 1] + b_ref[:, D:D + 1]
    o_ref[...] = fm + lin + lb_ref[...]


def _fm(rows_a, rows_b, lin_bias):
    return pl.pallas_call(
        _fm_body,
        in_specs=[
            pl.BlockSpec((B, W128), lambda: (0, 0)),
            pl.BlockSpec((B, W128), lambda: (0, 0)),
            pl.BlockSpec((1, 1), lambda: (0, 0)),
        ],
        out_specs=pl.BlockSpec((B, 1), lambda: (0, 0)),
        out_shape=jax.ShapeDtypeStruct((B, 1), jnp.float32),
    )(rows_a, rows_b, lin_bias)


def kernel(features, gcn_weight, gcn_bias, lin_table, lin_bias,
           interaction_pairs, edge_index):
    edges3 = edge_index.reshape(2, NCH, CHUNK)
    idx_a = interaction_pairs[:, 0].reshape(NW, PCH, CHUNK)
    idx_b = interaction_pairs[:, 1].reshape(NW, PCH, CHUNK)

    ones128 = jnp.ones((CHUNK,), jnp.float32)
    zeros_deg = jnp.zeros((NDEG,), jnp.float32)
    zeros_acc = jnp.zeros((NC, N, W128), jnp.float32)

    deg_parts = _deg_kernel(edges3, ones128, zeros_deg)
    deg_t = deg_parts.T

    x = _xw(features, gcn_weight)
    y = _make_y(x, deg_t[:N])

    acc_half = _edge_kernel0(y, edges3, zeros_acc)
    acc_parts = _edge_kernel1(y, edges3, acc_half)

    ext = _make_ext(acc_parts, y, deg_t[:N], gcn_bias.reshape(1, D), lin_table)

    rows_a, rows_b = _pair_gather_kernel(ext, idx_a, idx_b)

    out = _fm(rows_a, rows_b, lin_bias.reshape(1, 1))
    return out.reshape(B)
